# baseline (speedup 1.0000x reference)
import functools

import jax
import jax.numpy as jnp
from jax import lax
from jax.experimental import pallas as pl
from jax.experimental.pallas import tpu as pltpu
from jax.experimental.pallas import tpu_sc as plsc

NC = 2
NS = 16
L = 16


def kernel(word_inputs, embedding):
    B, S = word_inputs.shape
    V, H = embedding.shape
    NW = NC * NS
    BPW = B // NW

    mesh = plsc.VectorSubcoreMesh(
        core_axis_name="c", subcore_axis_name="s", num_cores=NC, num_subcores=NS
    )

    @functools.partial(
        pl.kernel,
        out_type=jax.ShapeDtypeStruct((S * B, H), jnp.float32),
        mesh=mesh,
        scratch_types=[
            pltpu.VMEM((BPW * S,), jnp.int32),
            pltpu.VMEM((BPW,), jnp.int32),
            pltpu.VMEM((BPW, H), jnp.float32),
            pltpu.SemaphoreType.DMA,
        ],
        compiler_params=pltpu.CompilerParams(
            use_tc_tiling_on_sc=False, needs_layout_passes=False
        ),
    )
    def emb_lookup(word_hbm, emb_hbm, out_hbm, idx_v, col_v, rows_v, sem):
        cid = lax.axis_index("c")
        sid = lax.axis_index("s")
        wid = sid * NC + cid
        b0 = wid * BPW
        pltpu.sync_copy(word_hbm.at[pl.ds(b0 * S, BPW * S)], idx_v)
        lane = lax.iota(jnp.int32, L)

        @pl.loop(0, S)
        def _(s):
            for j in range(BPW // L):
                flat_ids = (lane + (j * L)) * S + s
                col_v[pl.ds(j * L, L)] = plsc.load_gather(idx_v, [flat_ids])
            pltpu.async_copy(emb_hbm.at[col_v], rows_v, sem).wait()
            pltpu.sync_copy(rows_v, out_hbm.at[pl.ds(s * B + b0, BPW), :])

    out = emb_lookup(word_inputs.astype(jnp.int32).reshape(B * S), embedding)
    return out.reshape(S, B, H)


# speedup vs baseline: 1.1129x; 1.1129x over previous
import functools

import jax
import jax.numpy as jnp
from jax import lax
from jax.experimental import pallas as pl
from jax.experimental.pallas import tpu as pltpu
from jax.experimental.pallas import tpu_sc as plsc

NC = 2
NS = 16
L = 16
NBUF = 4


def kernel(word_inputs, embedding):
    B, S = word_inputs.shape
    V, H = embedding.shape
    NW = NC * NS
    BPW = B // NW
    assert S % NBUF == 0

    mesh = plsc.VectorSubcoreMesh(
        core_axis_name="c", subcore_axis_name="s", num_cores=NC, num_subcores=NS
    )

    @functools.partial(
        pl.kernel,
        out_type=jax.ShapeDtypeStruct((S * B, H), jnp.float32),
        mesh=mesh,
        scratch_types=(
            [pltpu.VMEM((BPW * S,), jnp.int32)]
            + [pltpu.VMEM((BPW,), jnp.int32)] * NBUF
            + [pltpu.VMEM((BPW, H), jnp.float32)] * NBUF
            + [pltpu.SemaphoreType.DMA] * (2 * NBUF)
        ),
        compiler_params=pltpu.CompilerParams(
            use_tc_tiling_on_sc=False, needs_layout_passes=False
        ),
    )
    def emb_lookup(word_hbm, emb_hbm, out_hbm, idx_v, *bufs):
        cols = bufs[:NBUF]
        rows = bufs[NBUF : 2 * NBUF]
        gsem = bufs[2 * NBUF : 3 * NBUF]
        wsem = bufs[3 * NBUF : 4 * NBUF]

        cid = lax.axis_index("c")
        sid = lax.axis_index("s")
        wid = sid * NC + cid
        b0 = wid * BPW
        pltpu.sync_copy(word_hbm.at[pl.ds(b0 * S, BPW * S)], idx_v)
        lane = lax.iota(jnp.int32, L)

        def build_col(i, b):
            for j in range(BPW // L):
                flat_ids = (lane + (j * L)) * S + i
                cols[b][pl.ds(j * L, L)] = plsc.load_gather(idx_v, [flat_ids])

        def wait_gather(b):
            pltpu.make_async_copy(emb_hbm.at[cols[b]], rows[b], gsem[b]).wait()

        def fire_write(i, b):
            pltpu.async_copy(rows[b], out_hbm.at[pl.ds(i * B + b0, BPW), :], wsem[b])

        def wait_write(b):
            pltpu.make_async_copy(
                rows[b], out_hbm.at[pl.ds(b0, BPW), :], wsem[b]
            ).wait()

        @pl.loop(0, S, step=NBUF)
        def _(g):
            for b in range(NBUF):
                i = g + b
                build_col(i, b)

                @pl.when(g + b >= NBUF)
                def _():
                    wait_write(b)

                pltpu.async_copy(emb_hbm.at[cols[b]], rows[b], gsem[b])
                pb = (b - 1) % NBUF

                @pl.when(g + b >= 1)
                def _():
                    wait_gather(pb)
                    fire_write(i - 1, pb)

        last = (S - 1) % NBUF
        wait_gather(last)
        fire_write(S - 1, last)
        for b in range(NBUF):
            wait_write(b)

    out = emb_lookup(word_inputs.astype(jnp.int32).reshape(B * S), embedding)
    return out.reshape(S, B, H)


# speedup vs baseline: 1.1134x; 1.0005x over previous
import functools

import jax
import jax.numpy as jnp
from jax import lax
from jax.experimental import pallas as pl
from jax.experimental.pallas import tpu as pltpu
from jax.experimental.pallas import tpu_sc as plsc

NC = 2
NS = 16
L = 16
NBUF = 4


def kernel(word_inputs, embedding):
    B, S = word_inputs.shape
    V, H = embedding.shape
    NW = NC * NS
    BPW = B // NW
    assert S % NBUF == 0

    mesh = plsc.VectorSubcoreMesh(
        core_axis_name="c", subcore_axis_name="s", num_cores=NC, num_subcores=NS
    )

    @functools.partial(
        pl.kernel,
        out_type=jax.ShapeDtypeStruct((S, B, H), jnp.float32),
        mesh=mesh,
        scratch_types=(
            [pltpu.VMEM((BPW, S), jnp.int32)]
            + [pltpu.VMEM((BPW,), jnp.int32)] * NBUF
            + [pltpu.VMEM((BPW, H), jnp.float32)] * NBUF
            + [pltpu.SemaphoreType.DMA] * (2 * NBUF)
        ),
        compiler_params=pltpu.CompilerParams(
            use_tc_tiling_on_sc=False, needs_layout_passes=False
        ),
    )
    def emb_lookup(word_hbm, emb_hbm, out_hbm, idx_v, *bufs):
        cols = bufs[:NBUF]
        rows = bufs[NBUF : 2 * NBUF]
        gsem = bufs[2 * NBUF : 3 * NBUF]
        wsem = bufs[3 * NBUF : 4 * NBUF]

        cid = lax.axis_index("c")
        sid = lax.axis_index("s")
        wid = sid * NC + cid
        b0 = wid * BPW
        pltpu.sync_copy(word_hbm.at[pl.ds(b0, BPW), :], idx_v)
        lane = lax.iota(jnp.int32, L)

        def build_col(i, b):
            col_ids = jnp.full((L,), 0, jnp.int32) + i
            for j in range(BPW // L):
                row_ids = lane + (j * L)
                cols[b][pl.ds(j * L, L)] = plsc.load_gather(
                    idx_v, [row_ids, col_ids]
                )

        def wait_gather(b):
            pltpu.make_async_copy(emb_hbm.at[cols[b]], rows[b], gsem[b]).wait()

        def fire_write(i, b):
            pltpu.async_copy(rows[b], out_hbm.at[i, pl.ds(b0, BPW), :], wsem[b])

        def wait_write(b):
            pltpu.make_async_copy(
                rows[b], out_hbm.at[0, pl.ds(b0, BPW), :], wsem[b]
            ).wait()

        @pl.loop(0, S, step=NBUF)
        def _(g):
            for b in range(NBUF):
                i = g + b
                build_col(i, b)

                @pl.when(g + b >= NBUF)
                def _():
                    wait_write(b)

                pltpu.async_copy(emb_hbm.at[cols[b]], rows[b], gsem[b])
                pb = (b - 1) % NBUF

                @pl.when(g + b >= 1)
                def _():
                    wait_gather(pb)
                    fire_write(i - 1, pb)

        last = (S - 1) % NBUF
        wait_gather(last)
        fire_write(S - 1, last)
        for b in range(NBUF):
            wait_write(b)

    return emb_lookup(word_inputs.astype(jnp.int32), embedding)


# speedup vs baseline: 1.3696x; 1.2301x over previous
import functools

import jax
import jax.numpy as jnp
from jax import lax
from jax.experimental import pallas as pl
from jax.experimental.pallas import tpu as pltpu
from jax.experimental.pallas import tpu_sc as plsc

NC = 2
NS = 16
L = 16
NBUF = 4
HP = 128


def kernel(word_inputs, embedding):
    B, S = word_inputs.shape
    V, H = embedding.shape
    NW = NC * NS
    BPW = B // NW
    assert S % NBUF == 0

    mesh = plsc.VectorSubcoreMesh(
        core_axis_name="c", subcore_axis_name="s", num_cores=NC, num_subcores=NS
    )

    @functools.partial(
        pl.kernel,
        out_type=jax.ShapeDtypeStruct((S * B, HP), jnp.float32),
        mesh=mesh,
        scratch_types=(
            [pltpu.VMEM((S, BPW), jnp.int32)]
            + [pltpu.VMEM((BPW, HP), jnp.float32)] * NBUF
            + [pltpu.SemaphoreType.DMA] * (2 * NBUF)
        ),
    )
    def emb_lookup(wordT_hbm, emb_hbm, out_hbm, idx_v, *bufs):
        rows = bufs[:NBUF]
        gsem = bufs[NBUF : 2 * NBUF]
        wsem = bufs[2 * NBUF : 3 * NBUF]

        cid = lax.axis_index("c")
        sid = lax.axis_index("s")
        wid = sid * NC + cid
        b0 = wid * BPW
        pltpu.sync_copy(wordT_hbm.at[:, pl.ds(b0, BPW)], idx_v)

        def wait_gather(b):
            pltpu.make_async_copy(emb_hbm.at[idx_v.at[0]], rows[b], gsem[b]).wait()

        def fire_write(i, b):
            pltpu.async_copy(
                rows[b], out_hbm.at[pl.ds(i * B + b0, BPW), :], wsem[b]
            )

        def wait_write(b):
            pltpu.make_async_copy(
                rows[b], out_hbm.at[pl.ds(b0, BPW), :], wsem[b]
            ).wait()

        @pl.loop(0, S, step=NBUF)
        def _(g):
            for b in range(NBUF):
                i = g + b

                @pl.when(g + b >= NBUF)
                def _():
                    wait_write(b)

                pltpu.async_copy(emb_hbm.at[idx_v.at[i]], rows[b], gsem[b])
                pb = (b - 1) % NBUF

                @pl.when(g + b >= 1)
                def _():
                    wait_gather(pb)
                    fire_write(i - 1, pb)

        last = (S - 1) % NBUF
        wait_gather(last)
        fire_write(S - 1, last)
        for b in range(NBUF):
            wait_write(b)

    wordT = jnp.transpose(word_inputs.astype(jnp.int32))
    emb_padded = jnp.pad(embedding, ((0, 0), (0, HP - H)))
    out = emb_lookup(wordT, emb_padded)
    return out[:, :H].reshape(S, B, H)
